# baseline (speedup 1.0000x reference)
import functools

import jax
import jax.numpy as jnp
from jax import lax
from jax.experimental import pallas as pl
from jax.experimental.pallas import tpu as pltpu
from jax.experimental.pallas import tpu_sc as plsc

_B, _H, _W = 16, 512, 512
_CARD = 50
_N = _B * _CARD * 2
_PER_TILE = 64
_ACTIVE = _N // _PER_TILE
_LANES = 16


def _sc_gather(x_flat, ids0, ids1, mask_i32):
    mesh = plsc.VectorSubcoreMesh(core_axis_name="c", subcore_axis_name="s")
    info = plsc.get_sparse_core_info()
    num_cores = info.num_cores

    @functools.partial(
        pl.kernel,
        mesh=mesh,
        out_type=jax.ShapeDtypeStruct((_N,), jnp.float32),
        scratch_types=[
            pltpu.VMEM((_PER_TILE,), jnp.int32),
            pltpu.VMEM((_PER_TILE,), jnp.int32),
            pltpu.VMEM((_PER_TILE,), jnp.int32),
            pltpu.VMEM((_PER_TILE,), jnp.int32),
            pltpu.VMEM((_PER_TILE,), jnp.float32),
            pltpu.SemaphoreType.DMA,
        ],
    )
    def body(x_hbm, i0_hbm, i1_hbm, m_hbm, out_hbm,
             i0_v, i1_v, m_v, idx_v, vals_v, sem):
        wid = lax.axis_index("s") * num_cores + lax.axis_index("c")

        @pl.when(wid < _ACTIVE)
        def _():
            base = wid * _PER_TILE
            pltpu.sync_copy(i0_hbm.at[pl.ds(base, _PER_TILE)], i0_v)
            pltpu.sync_copy(i1_hbm.at[pl.ds(base, _PER_TILE)], i1_v)
            pltpu.sync_copy(m_hbm.at[pl.ds(base, _PER_TILE)], m_v)
            for i in range(_PER_TILE // _LANES):
                s = pl.ds(i * _LANES, _LANES)
                idx_v[s] = i0_v[s] * _W + i1_v[s]
            pltpu.async_copy(x_hbm.at[idx_v], vals_v, sem).wait()
            for i in range(_PER_TILE // _LANES):
                s = pl.ds(i * _LANES, _LANES)
                vals_v[s] = jnp.where(m_v[s] != 0, jnp.float32(0.0), vals_v[s])
            pltpu.sync_copy(vals_v, out_hbm.at[pl.ds(base, _PER_TILE)])

    return body(x_flat, ids0, ids1, mask_i32)


def kernel(x, ids0, ids1, ids_mask):
    x_flat = x.reshape(-1)
    flat = _sc_gather(
        x_flat,
        ids0.reshape(_N),
        ids1.reshape(_N),
        ids_mask.reshape(_N).astype(jnp.int32),
    )
    return flat.reshape(_B, _CARD, 2)


# speedup vs baseline: 1.2572x; 1.2572x over previous
import functools

import jax
import jax.numpy as jnp
from jax import lax
from jax.experimental import pallas as pl
from jax.experimental.pallas import tpu as pltpu
from jax.experimental.pallas import tpu_sc as plsc

_B, _H, _W = 16, 512, 512
_CARD = 50
_N = _B * _CARD * 2
_PER_TILE = 64
_ACTIVE = _N // _PER_TILE
_LANES = 16


def _sc_gather(x2d, packed):
    mesh = plsc.VectorSubcoreMesh(core_axis_name="c", subcore_axis_name="s")
    info = plsc.get_sparse_core_info()
    num_cores = info.num_cores

    @functools.partial(
        pl.kernel,
        mesh=mesh,
        out_type=(
            jax.ShapeDtypeStruct((_N,), jnp.float32),
            jax.ShapeDtypeStruct((_N * _W,), jnp.float32),
        ),
        scratch_types=[
            pltpu.VMEM((_PER_TILE,), jnp.int32),
            pltpu.VMEM((_PER_TILE,), jnp.int32),
            pltpu.VMEM((_PER_TILE,), jnp.int32),
            pltpu.VMEM((_PER_TILE,), jnp.int32),
            pltpu.VMEM((_PER_TILE, _W), jnp.float32),
            pltpu.VMEM((_PER_TILE,), jnp.float32),
            pltpu.SemaphoreType.DMA,
            pltpu.SemaphoreType.DMA,
        ],
    )
    def body(x_hbm, p_hbm, out_hbm, stage_hbm,
             p_v, row_v, eidx_v, m_v, rows_v, vals_v, sem, wsem):
        wid = lax.axis_index("s") * num_cores + lax.axis_index("c")

        @pl.when(wid < _ACTIVE)
        def _():
            base = wid * _PER_TILE
            pltpu.sync_copy(p_hbm.at[pl.ds(base, _PER_TILE)], p_v)
            for i in range(_PER_TILE // _LANES):
                s = pl.ds(i * _LANES, _LANES)
                j = lax.iota(jnp.int32, _LANES) + jnp.int32(i * _LANES)
                c = p_v[s]
                row_v[s] = c >> 10
                col = (c >> 1) & jnp.int32(_W - 1)
                eidx_v[s] = (base + j) * _W + col
                m_v[s] = c & 1
            pltpu.async_copy(x_hbm.at[row_v], rows_v, sem).wait()
            for j in range(_PER_TILE):
                for k in range(_W // 128):
                    pltpu.async_copy(
                        rows_v.at[j, pl.ds(k * 128, 128)],
                        stage_hbm.at[pl.ds((base + j) * _W + k * 128, 128)],
                        wsem,
                    )
            for j in range(_PER_TILE):
                for k in range(_W // 128):
                    pltpu.make_async_copy(
                        rows_v.at[j, pl.ds(k * 128, 128)],
                        stage_hbm.at[pl.ds((base + j) * _W + k * 128, 128)],
                        wsem,
                    ).wait()
            pltpu.async_copy(stage_hbm.at[eidx_v], vals_v, sem).wait()
            for i in range(_PER_TILE // _LANES):
                s = pl.ds(i * _LANES, _LANES)
                vals_v[s] = jnp.where(m_v[s] != 0, jnp.float32(0.0),
                                      vals_v[s])
            pltpu.sync_copy(vals_v, out_hbm.at[pl.ds(base, _PER_TILE)])

    return body(x2d, packed)


def kernel(x, ids0, ids1, ids_mask):
    x2d = x.reshape(_B * _H, _W)
    packed = (
        (ids0 << 10) | (ids1 << 1) | ids_mask.astype(jnp.int32)
    ).reshape(_N)
    flat, _ = _sc_gather(x2d, packed)
    return flat.reshape(_B, _CARD, 2)
